# baseline (speedup 1.0000x reference)
import functools

import jax
import jax.numpy as jnp
from jax import lax
from jax.experimental import pallas as pl
from jax.experimental.pallas import tpu as pltpu
from jax.experimental.pallas import tpu_sc as plsc

NUM_NEG = 999
ROW = NUM_NEG + 1
USERS = 8192
TOP_K = 10
FMIN = float(jnp.finfo(jnp.float32).min)

NW = 32
RPW = USERS // NW
RB = 16
NB = RPW // RB
GB = RB * ROW
TPB = GB // 128
NROWS = 128000


def _sc_body(x_hbm, dup_hbm, itk_hbm, mw_hbm,
             xbuf0, xbuf1, dbuf0, dbuf1, ibuf0, ibuf1, itkbuf, mwbuf,
             semx0, semx1, semd0, semd1):
    ncores = 2
    wid = lax.axis_index("s") * ncores + lax.axis_index("c")
    lanes = lax.iota(jnp.int32, 16)
    row_g = lanes * ROW
    fmin = jnp.full((16,), FMIN, jnp.float32)
    semx = (semx0, semx1)
    semd = (semd0, semd1)
    xbufs = (xbuf0, xbuf1)
    dbufs = (dbuf0, dbuf1)
    ibufs = (ibuf0, ibuf1)

    copies = {}

    def start(b):
        slot = b % 2
        g0 = (wid * RPW + b * RB) * ROW
        t0 = g0 // 128
        ib = ibufs[slot]
        for v in range(8):
            k = jnp.minimum(16 * v + lanes, TPB - 1)
            ib[pl.ds(16 * v, 16)] = 2 * (t0 + k) + 1
        cx = pltpu.async_copy(x_hbm.at[ib], xbufs[slot], semx[slot])
        cd = pltpu.async_copy(
            dup_hbm.at[pl.ds(g0, GB)], dbufs[slot], semd[slot])
        copies[b] = (cx, cd)

    start(0)
    for b in range(NB):
        if b + 1 < NB:
            start(b + 1)
        cx, cd = copies.pop(b)
        cx.wait()
        cd.wait()
        slot = b % 2
        xb = xbufs[slot]
        db = dbufs[slot]

        g999 = row_g + NUM_NEG
        t_x = plsc.load_gather(xb, [g999 >> 7, g999 & 127])
        d999 = plsc.load_gather(db, [g999])
        t = jnp.where(d999 == 1, fmin, t_x)
        tmin = (t <= fmin).astype(jnp.int32)

        def step(j, c):
            cnt, dsum = c
            g = row_g + j
            x = plsc.load_gather(xb, [g >> 7, g & 127])
            d = plsc.load_gather(db, [g])
            ge = (x >= t).astype(jnp.int32)
            cnt = cnt + jnp.where(d == 1, tmin, ge)
            return cnt, dsum + d

        zero = jnp.zeros((16,), jnp.int32)
        cnt, dsum = plsc.parallel_loop(
            0, NUM_NEG, unroll=8, carry=(zero, zero))(step)

        itk = (cnt < TOP_K).astype(jnp.float32)
        mw = ((dsum + d999) != NUM_NEG).astype(jnp.float32)
        itkbuf[pl.ds(b * RB, RB)] = itk
        mwbuf[pl.ds(b * RB, RB)] = mw

    pltpu.sync_copy(itkbuf, itk_hbm.at[pl.ds(wid * RPW, RPW)])
    pltpu.sync_copy(mwbuf, mw_hbm.at[pl.ds(wid * RPW, RPW)])


_sc_metric = pl.kernel(
    _sc_body,
    out_type=(
        jax.ShapeDtypeStruct((USERS,), jnp.float32),
        jax.ShapeDtypeStruct((USERS,), jnp.float32),
    ),
    mesh=plsc.VectorSubcoreMesh(core_axis_name="c", subcore_axis_name="s"),
    compiler_params=pltpu.CompilerParams(needs_layout_passes=False),
    scratch_types=[
        pltpu.VMEM((128, 128), jnp.float32),
        pltpu.VMEM((128, 128), jnp.float32),
        pltpu.VMEM((GB,), jnp.int32),
        pltpu.VMEM((GB,), jnp.int32),
        pltpu.VMEM((128,), jnp.int32),
        pltpu.VMEM((128,), jnp.int32),
        pltpu.VMEM((RPW,), jnp.float32),
        pltpu.VMEM((RPW,), jnp.float32),
        pltpu.SemaphoreType.DMA,
        pltpu.SemaphoreType.DMA,
        pltpu.SemaphoreType.DMA,
        pltpu.SemaphoreType.DMA,
    ],
)


def _copy_body(src_ref, dst_ref):
    dst_ref[...] = src_ref[...]


_tc_copy = pl.pallas_call(
    _copy_body,
    out_shape=jax.ShapeDtypeStruct((NROWS, 128), jnp.float32),
    grid=(64,),
    in_specs=[pl.BlockSpec((2000, 128), lambda i: (i, 0))],
    out_specs=pl.BlockSpec((2000, 128), lambda i: (i, 0)),
)


def _hr_body(itk_ref, mw_ref, hr_ref):
    itk = itk_ref[...]
    mw = mw_ref[...]
    num = jnp.sum(itk * mw)
    den = jnp.maximum(jnp.sum(mw), 1e-9)
    hr_ref[0, 0] = num / den


_hr_reduce = pl.pallas_call(
    _hr_body,
    out_shape=jax.ShapeDtypeStruct((1, 1), jnp.float32),
    in_specs=[
        pl.BlockSpec(memory_space=pltpu.VMEM),
        pl.BlockSpec(memory_space=pltpu.VMEM),
    ],
    out_specs=pl.BlockSpec(memory_space=pltpu.SMEM),
)


def kernel(logits, dup_mask):
    x2d = (logits.reshape(64000, 128, 2).transpose(0, 2, 1)
           .reshape(NROWS, 128))
    dup_flat = dup_mask.reshape(-1)
    itk, mw = _sc_metric(x2d, dup_flat)
    hr = _hr_reduce(itk.reshape(64, 128), mw.reshape(64, 128))[0, 0]
    out_flat = _tc_copy(x2d)
    out_logits = (out_flat.reshape(64000, 2, 128)
                  .transpose(0, 2, 1).reshape(8192000, 1, 2))
    return out_logits, itk, mw, hr


# speedup vs baseline: 1.0872x; 1.0872x over previous
import functools

import jax
import jax.numpy as jnp
from jax import lax
from jax.experimental import pallas as pl
from jax.experimental.pallas import tpu as pltpu
from jax.experimental.pallas import tpu_sc as plsc

NUM_NEG = 999
ROW = NUM_NEG + 1
USERS = 8192
TOP_K = 10
FMIN = float(jnp.finfo(jnp.float32).min)

NW = 32
RPW = USERS // NW
RB = 16
NB = RPW // RB
GB = RB * ROW
TPB = GB // 128
NROWS = 128000


def _sc_body(x_hbm, dup_hbm, itk_hbm, mw_hbm,
             xbuf0, xbuf1, dbuf0, dbuf1, ibuf0, ibuf1, itkbuf, mwbuf,
             semx0, semx1, semd0, semd1):
    ncores = 2
    wid = lax.axis_index("s") * ncores + lax.axis_index("c")
    lanes = lax.iota(jnp.int32, 16)
    row_g = lanes * ROW
    fmin = jnp.full((16,), FMIN, jnp.float32)
    semx = (semx0, semx1)
    semd = (semd0, semd1)
    xbufs = (xbuf0, xbuf1)
    dbufs = (dbuf0, dbuf1)
    ibufs = (ibuf0, ibuf1)

    copies = {}

    def start(b):
        slot = b % 2
        g0 = (wid * RPW + b * RB) * ROW
        t0 = g0 // 128
        ib = ibufs[slot]
        for v in range(8):
            k = jnp.minimum(16 * v + lanes, TPB - 1)
            ib[pl.ds(16 * v, 16)] = 2 * (t0 + k) + 1
        cx = pltpu.async_copy(x_hbm.at[ib], xbufs[slot], semx[slot])
        cd = pltpu.async_copy(
            dup_hbm.at[pl.ds(g0, GB)], dbufs[slot], semd[slot])
        copies[b] = (cx, cd)

    start(0)
    for b in range(NB):
        if b + 1 < NB:
            start(b + 1)
        cx, cd = copies.pop(b)
        cx.wait()
        cd.wait()
        slot = b % 2
        xb = xbufs[slot]
        db = dbufs[slot]

        g999 = row_g + NUM_NEG
        t_x = plsc.load_gather(xb, [g999 >> 7, g999 & 127])
        d999 = plsc.load_gather(db, [g999])
        t = jnp.where(d999 == 1, fmin, t_x)
        tmin = (t <= fmin).astype(jnp.int32)

        def step(j, c):
            cnt, dsum = c
            g = row_g + j
            x = plsc.load_gather(xb, [g >> 7, g & 127])
            d = plsc.load_gather(db, [g])
            ge = (x >= t).astype(jnp.int32)
            cnt = cnt + jnp.where(d == 1, tmin, ge)
            return cnt, dsum + d

        zero = jnp.zeros((16,), jnp.int32)
        cnt, dsum = plsc.parallel_loop(
            0, NUM_NEG, unroll=8, carry=(zero, zero))(step)

        itk = (cnt < TOP_K).astype(jnp.float32)
        mw = ((dsum + d999) != NUM_NEG).astype(jnp.float32)
        itkbuf[pl.ds(b * RB, RB)] = itk
        mwbuf[pl.ds(b * RB, RB)] = mw

    pltpu.sync_copy(itkbuf, itk_hbm.at[pl.ds(wid * RPW, RPW)])
    pltpu.sync_copy(mwbuf, mw_hbm.at[pl.ds(wid * RPW, RPW)])


_sc_metric = pl.kernel(
    _sc_body,
    out_type=(
        jax.ShapeDtypeStruct((USERS,), jnp.float32),
        jax.ShapeDtypeStruct((USERS,), jnp.float32),
    ),
    mesh=plsc.VectorSubcoreMesh(core_axis_name="c", subcore_axis_name="s"),
    compiler_params=pltpu.CompilerParams(needs_layout_passes=False),
    scratch_types=[
        pltpu.VMEM((128, 128), jnp.float32),
        pltpu.VMEM((128, 128), jnp.float32),
        pltpu.VMEM((GB,), jnp.int32),
        pltpu.VMEM((GB,), jnp.int32),
        pltpu.VMEM((128,), jnp.int32),
        pltpu.VMEM((128,), jnp.int32),
        pltpu.VMEM((RPW,), jnp.float32),
        pltpu.VMEM((RPW,), jnp.float32),
        pltpu.SemaphoreType.DMA,
        pltpu.SemaphoreType.DMA,
        pltpu.SemaphoreType.DMA,
        pltpu.SemaphoreType.DMA,
    ],
)


def _copy_body(src_ref, dst_ref):
    dst_ref[...] = src_ref[...]


_tc_copy = pl.pallas_call(
    _copy_body,
    out_shape=jax.ShapeDtypeStruct((NROWS, 128), jnp.float32),
    grid=(32,),
    in_specs=[pl.BlockSpec((4000, 128), lambda i: (i, 0))],
    out_specs=pl.BlockSpec((4000, 128), lambda i: (i, 0)),
)


def _hr_body(itk_ref, mw_ref, hr_ref):
    itk = itk_ref[...]
    mw = mw_ref[...]
    num = jnp.sum(itk * mw)
    den = jnp.maximum(jnp.sum(mw), 1e-9)
    hr_ref[0, 0] = num / den


_hr_reduce = pl.pallas_call(
    _hr_body,
    out_shape=jax.ShapeDtypeStruct((1, 1), jnp.float32),
    in_specs=[
        pl.BlockSpec(memory_space=pltpu.VMEM),
        pl.BlockSpec(memory_space=pltpu.VMEM),
    ],
    out_specs=pl.BlockSpec(memory_space=pltpu.SMEM),
)


def kernel(logits, dup_mask):
    x2d = (logits.reshape(64000, 128, 2).transpose(0, 2, 1)
           .reshape(NROWS, 128))
    dup_flat = dup_mask.reshape(-1)
    itk, mw = _sc_metric(x2d, dup_flat)
    hr = _hr_reduce(itk.reshape(64, 128), mw.reshape(64, 128))[0, 0]
    out_flat = _tc_copy(x2d)
    out_logits = (out_flat.reshape(64000, 2, 128)
                  .transpose(0, 2, 1).reshape(8192000, 1, 2))
    return out_logits, itk, mw, hr
